# baseline (speedup 1.0000x reference)
import functools

import jax
import jax.numpy as jnp
from jax import lax
from jax.experimental import pallas as pl
from jax.experimental.pallas import tpu as pltpu
from jax.experimental.pallas import tpu_sc as plsc

H = 256
N_NAME = 10000
N_ATTR = 1000
L = 8192
NEG = 0.01
EPS = 1e-5
NR = 10240
BN = 1024
NBLK = NR // BN

E = 160000
NS = 16
EW = E // NS
IDXR = 79
CW = 1024
CH = 1280
NCH = 8
WR = CH // NS
WEL = WR * CW
ROWPAD = NCH * CH
DUMP = CH * CW
NDUMP = 2048
CBUF = DUMP + NDUMP
CNTW = CW * NS
CNTBUF = CNTW + NDUMP
ZB = 4096


def _leaky(x):
    return jnp.where(x >= 0, x, NEG * x)


def _dot(a, b):
    return jnp.dot(a, b, preferred_element_type=jnp.float32)


def _build_counts(dst_a2n, src_a2n, src_n2a, dst_n2a):
    mesh = plsc.VectorSubcoreMesh(core_axis_name="c", subcore_axis_name="s")

    import functools as _ft

    @_ft.partial(
        pl.kernel,
        out_type=[
            jax.ShapeDtypeStruct((ROWPAD * CW,), jnp.float32),
            jax.ShapeDtypeStruct((ROWPAD * CW,), jnp.float32),
            jax.ShapeDtypeStruct((CNTW,), jnp.float32),
        ],
        mesh=mesh,
        scratch_types=[
            pltpu.VMEM((EW,), jnp.int32),
            pltpu.VMEM((EW,), jnp.int32),
            pltpu.VMEM((IDXR * 128,), jnp.int32),
            pltpu.VMEM((IDXR * 128,), jnp.float32),
            pltpu.VMEM((ZB,), jnp.float32),
            pltpu.VMEM_SHARED((CBUF,), jnp.float32),
            pltpu.VMEM_SHARED((CNTBUF,), jnp.float32),
            pltpu.SemaphoreType.DMA,
        ],
    )
    def k(d_a2n_h, s_a2n_h, s_n2a_h, d_n2a_h, out_a, out_b, out_cnt,
          rows_v, cols_v, idx1, ones2, zbuf, cbuf, cntbuf, sem):
        cid = lax.axis_index("c")
        sid = lax.axis_index("s")
        zero16 = jnp.zeros((16,), jnp.float32)

        def zinit(i, c):
            zbuf[pl.ds(i * 16, 16)] = zero16
            return c
        lax.fori_loop(0, ZB // 16, zinit, 0)
        one16 = jnp.full((16,), 1.0, jnp.float32)

        def oinit(r, c):
            ones2[pl.ds(r * 16, 16)] = one16
            return c
        lax.fori_loop(0, IDXR * 8, oinit, 0)

        for rows_h, cols_h, out in ((d_a2n_h, s_a2n_h, out_a),
                                    (s_n2a_h, d_n2a_h, out_b)):
            with jax.named_scope("edge_stage"):
                pltpu.sync_copy(rows_h.at[pl.ds(sid * EW, EW)], rows_v)
                pltpu.sync_copy(cols_h.at[pl.ds(sid * EW, EW)], cols_v)
            for p in range(NCH // 2):
                lo = (2 * p + cid) * CH
                hi = lo + CH
                base = sid * WEL
                with jax.named_scope("zero_chunk"):
                    nz = WEL // ZB
                    hz = [pltpu.async_copy(
                        zbuf.at[pl.ds(0, ZB)],
                        cbuf.at[pl.ds(base + kk * ZB, ZB)], sem)
                          for kk in range(nz)]
                    tail = WEL - nz * ZB
                    if tail:
                        hz.append(pltpu.async_copy(
                            zbuf.at[pl.ds(0, tail)],
                            cbuf.at[pl.ds(base + nz * ZB, tail)], sem))
                    for h in hz:
                        h.wait()

                with jax.named_scope("build_idx"):
                    iota16 = lax.iota(jnp.int32, 16)

                    def build(r, c):
                        e = r * 16
                        d = rows_v[pl.ds(e, 16)]
                        s = cols_v[pl.ds(e, 16)]
                        dmp = DUMP + (e & (NDUMP - 1)) + iota16
                        f = jnp.where((d >= lo) & (d < hi),
                                      (d - lo) * CW + s, dmp)
                        idx1[pl.ds(e, 16)] = f
                        return c
                    lax.fori_loop(0, EW // 16, build, 0)
                    for t in range(EW // 16, IDXR * 8):
                        idx1[pl.ds(t * 16, 16)] = (
                            DUMP + ((t * 16) & (NDUMP - 1)) + iota16)

                plsc.subcore_barrier()
                with jax.named_scope("scatter"):
                    pltpu.sync_copy(ones2, cbuf.at[idx1], add=True)
                plsc.subcore_barrier()
                with jax.named_scope("writeback"):
                    pltpu.sync_copy(cbuf.at[pl.ds(base, WEL)],
                                    out.at[pl.ds(lo * CW + base, WEL)])

        pltpu.sync_copy(d_n2a_h.at[pl.ds(sid * EW, EW)], rows_v)
        zc = CNTBUF // NS
        pltpu.sync_copy(zbuf.at[pl.ds(0, zc)],
                        cntbuf.at[pl.ds(sid * zc, zc)])
        iota16c = lax.iota(jnp.int32, 16)

        def build_cnt(r, c):
            e = r * 16
            idx1[pl.ds(e, 16)] = rows_v[pl.ds(e, 16)] * NS + sid
            return c
        lax.fori_loop(0, EW // 16, build_cnt, 0)
        for t in range(EW // 16, IDXR * 8):
            idx1[pl.ds(t * 16, 16)] = (
                CNTW + ((t * 16) & (NDUMP - 1)) + iota16c)

        plsc.subcore_barrier()
        with jax.named_scope("scatter_cnt"):
            pltpu.sync_copy(ones2, cntbuf.at[idx1], add=True)
        plsc.subcore_barrier()

        @pl.when(jnp.logical_and(cid == 0, sid == 0))
        def _():
            pltpu.sync_copy(cntbuf.at[pl.ds(0, CNTW)], out_cnt)

    return k(dst_a2n, src_a2n, src_n2a, dst_n2a)


def _tdot(ct, h):
    return lax.dot_general(ct, h, (((0,), (0,)), ((), ())),
                           preferred_element_type=jnp.float32)


def _enc_name_body(g, w0, b0, w1, b1, ct, x_out, a_out, acc_a):
    i = pl.program_id(0)
    h = _leaky(_dot(g[...], w0[...]) + b0[...])
    h = _leaky(_dot(h, w1[...]) + b1[...])
    x_out[...] = h

    @pl.when(i == 0)
    def _():
        acc_a[...] = jnp.zeros_like(acc_a)

    acc_a[...] += _tdot(ct[...].astype(jnp.float32), h)

    @pl.when(i == NBLK - 1)
    def _():
        a_out[...] = acc_a[...]


def _enc_name(g_n, w0, b0, w1, b1, c_n2a_t):
    return pl.pallas_call(
        _enc_name_body,
        grid=(NBLK,),
        in_specs=[
            pl.BlockSpec((BN, H), lambda i: (i, 0)),
            pl.BlockSpec((H, H), lambda i: (0, 0)),
            pl.BlockSpec((1, H), lambda i: (0, 0)),
            pl.BlockSpec((H, H), lambda i: (0, 0)),
            pl.BlockSpec((1, H), lambda i: (0, 0)),
            pl.BlockSpec((BN, CW), lambda i: (i, 0)),
        ],
        out_specs=[
            pl.BlockSpec((BN, H), lambda i: (i, 0)),
            pl.BlockSpec((CW, H), lambda i: (0, 0)),
        ],
        out_shape=[
            jax.ShapeDtypeStruct((NR, H), jnp.float32),
            jax.ShapeDtypeStruct((CW, H), jnp.float32),
        ],
        scratch_shapes=[
            pltpu.VMEM((CW, H), jnp.float32),
        ],
    )(g_n, w0, b0, w1, b1, c_n2a_t)


def _attr_stage_body(with_enc, with_leaky, *refs):
    if with_enc:
        (g, w0, b0, w1, b1, agg, cnt, ws, wn, bb, gamma, beta, x_enc_out,
         x_out) = refs
        h = _leaky(_dot(g[...], w0[...]) + b0[...])
        h = _leaky(_dot(h, w1[...]) + b1[...])
        x_enc_out[...] = h
    else:
        (g, agg, cnt, ws, wn, bb, gamma, beta, x_out) = refs
        h = g[...]
    cn = jnp.sum(cnt[...], axis=1, keepdims=True)[:N_ATTR]
    aggr = agg[...] / jnp.maximum(cn, 1.0)
    pre = _dot(h, ws[...]) + _dot(aggr, wn[...]) + bb[...]
    mu = jnp.mean(pre, axis=0, keepdims=True)
    var = jnp.mean((pre - mu) ** 2, axis=0, keepdims=True)
    y = (pre - mu) * lax.rsqrt(var + EPS) * gamma[...] + beta[...]
    if with_leaky:
        y = _leaky(y)
    x_out[...] = y


def _attr_stage(with_enc, with_leaky, args):
    n_in = len(args)
    n_out = 2 if with_enc else 1
    full = lambda s: pl.BlockSpec(s, lambda: (0, 0))
    in_specs = [full(a.shape) for a in args]
    return pl.pallas_call(
        functools.partial(_attr_stage_body, with_enc, with_leaky),
        grid=(),
        in_specs=in_specs,
        out_specs=[full((N_ATTR, H))] * n_out,
        out_shape=[jax.ShapeDtypeStruct((N_ATTR, H), jnp.float32)] * n_out,
    )(*args)


def _name_conv_body(x, c, xa, ws, wn, bb, pre_out, stats_out, s1, s2):
    i = pl.program_id(0)
    cb = c[...].astype(jnp.float32)
    rs = jnp.sum(cb, axis=1, keepdims=True)
    aggr = _dot(cb, xa[...]) / jnp.maximum(rs, 1.0)
    pre = _dot(x[...], ws[...]) + _dot(aggr, wn[...]) + bb[...]
    pre_out[...] = pre

    @pl.when(i == 0)
    def _():
        s1[...] = jnp.zeros_like(s1)
        s2[...] = jnp.zeros_like(s2)

    row = i * BN + lax.broadcasted_iota(jnp.int32, (BN, 1), 0)
    pm = jnp.where(row < N_NAME, pre, 0.0)
    s1[...] += jnp.sum(pm, axis=0, keepdims=True)
    s2[...] += jnp.sum(pm * pm, axis=0, keepdims=True)

    @pl.when(i == NBLK - 1)
    def _():
        stats_out[0:1, :] = s1[...]
        stats_out[1:2, :] = s2[...]


def _name_conv(x_n, c_a2n, x_a, ws, wn, bb):
    return pl.pallas_call(
        _name_conv_body,
        grid=(NBLK,),
        in_specs=[
            pl.BlockSpec((BN, H), lambda i: (i, 0)),
            pl.BlockSpec((BN, CW), lambda i: (i, 0)),
            pl.BlockSpec((CW, H), lambda i: (0, 0)),
            pl.BlockSpec((H, H), lambda i: (0, 0)),
            pl.BlockSpec((H, H), lambda i: (0, 0)),
            pl.BlockSpec((1, H), lambda i: (0, 0)),
        ],
        out_specs=[
            pl.BlockSpec((BN, H), lambda i: (i, 0)),
            pl.BlockSpec((2, H), lambda i: (0, 0)),
        ],
        out_shape=[
            jax.ShapeDtypeStruct((NR, H), jnp.float32),
            jax.ShapeDtypeStruct((2, H), jnp.float32),
        ],
        scratch_shapes=[
            pltpu.VMEM((1, H), jnp.float32),
            pltpu.VMEM((1, H), jnp.float32),
        ],
    )(x_n, c_a2n, x_a, ws, wn, bb)


def _bn_accum_body(pre, stats, gamma, beta, c, x_out, a_out, acc):
    i = pl.program_id(0)
    mu = stats[0:1, :] / N_NAME
    var = stats[1:2, :] / N_NAME - mu * mu
    y = (pre[...] - mu) * lax.rsqrt(var + EPS) * gamma[...] + beta[...]
    y = _leaky(y)
    x_out[...] = y

    @pl.when(i == 0)
    def _():
        acc[...] = jnp.zeros_like(acc)

    acc[...] += _tdot(c[...].astype(jnp.float32), y)

    @pl.when(i == NBLK - 1)
    def _():
        a_out[...] = acc[...]


def _bn_accum(pre_n, stats, gamma, beta, c_n2a_t):
    return pl.pallas_call(
        _bn_accum_body,
        grid=(NBLK,),
        in_specs=[
            pl.BlockSpec((BN, H), lambda i: (i, 0)),
            pl.BlockSpec((2, H), lambda i: (0, 0)),
            pl.BlockSpec((1, H), lambda i: (0, 0)),
            pl.BlockSpec((1, H), lambda i: (0, 0)),
            pl.BlockSpec((BN, CW), lambda i: (i, 0)),
        ],
        out_specs=[
            pl.BlockSpec((BN, H), lambda i: (i, 0)),
            pl.BlockSpec((CW, H), lambda i: (0, 0)),
        ],
        out_shape=[
            jax.ShapeDtypeStruct((NR, H), jnp.float32),
            jax.ShapeDtypeStruct((CW, H), jnp.float32),
        ],
        scratch_shapes=[pltpu.VMEM((CW, H), jnp.float32)],
    )(pre_n, stats, gamma, beta, c_n2a_t)


LB = 1024
LBLK = L // LB


def _distmult_body(rows, stats, gamma, beta, xa, out):
    mu = stats[0:1, :] / N_NAME
    var = stats[1:2, :] / N_NAME - mu * mu
    y = (rows[...] - mu) * lax.rsqrt(var + EPS) * gamma[...] + beta[...]
    out[...] = lax.dot_general(y, xa[...], (((1,), (1,)), ((), ())),
                               preferred_element_type=jnp.float32)


def _distmult(rows, stats, gamma, beta, x_a):
    return pl.pallas_call(
        _distmult_body,
        grid=(LBLK,),
        in_specs=[
            pl.BlockSpec((LB, H), lambda i: (i, 0)),
            pl.BlockSpec((2, H), lambda i: (0, 0)),
            pl.BlockSpec((1, H), lambda i: (0, 0)),
            pl.BlockSpec((1, H), lambda i: (0, 0)),
            pl.BlockSpec((N_ATTR, H), lambda i: (0, 0)),
        ],
        out_specs=pl.BlockSpec((LB, N_ATTR), lambda i: (i, 0)),
        out_shape=jax.ShapeDtypeStruct((L, N_ATTR), jnp.float32),
    )(rows, stats, gamma, beta, x_a)


def kernel(params, node_feature_name, node_feature_attr, edge_src_n2a,
           edge_dst_n2a, edge_src_a2n, edge_dst_a2n, edge_label_src,
           edge_label_dst, node_label_attr):
    p = params
    r = lambda v: jnp.reshape(v, (1, H))

    idx_n = jnp.concatenate(
        [node_feature_name[:, 0], jnp.zeros((NR - N_NAME,), jnp.int32)])
    g_n = jnp.take(p['emb_name'], idx_n, axis=0)
    g_a = jnp.take(p['emb_attr'], node_feature_attr[:, 0], axis=0)
    ca_flat, cbt_flat, cnt_raw = _build_counts(
        edge_dst_a2n, edge_src_a2n, edge_src_n2a, edge_dst_n2a)
    c_a2n = jnp.reshape(ca_flat, (ROWPAD, CW))
    c_n2a_t = jnp.reshape(cbt_flat, (ROWPAD, CW))
    cnt_a = jnp.reshape(cnt_raw, (CW, NS))
    pad_a = lambda v: jnp.pad(v, ((0, CW - N_ATTR), (0, 0)))

    x0_n, a0_a = _enc_name(
        g_n, p['mlp_name_W0'], r(p['mlp_name_b0']),
        p['mlp_name_W1'], r(p['mlp_name_b1']), c_n2a_t)
    x0_a, x1_a = _attr_stage(True, True, (
        g_a, p['mlp_attr_W0'], r(p['mlp_attr_b0']),
        p['mlp_attr_W1'], r(p['mlp_attr_b1']),
        a0_a[:N_ATTR], cnt_a,
        p['conv0_n2a_Wself'], p['conv0_n2a_Wneigh'], r(p['conv0_n2a_b']),
        r(p['bn0_attr_gamma']), r(p['bn0_attr_beta'])))
    pre_n1, stats1 = _name_conv(
        x0_n, c_a2n, pad_a(x0_a),
        p['conv0_a2n_Wself'], p['conv0_a2n_Wneigh'], r(p['conv0_a2n_b']))
    x1_n, a1_a = _bn_accum(pre_n1, stats1, r(p['bn0_name_gamma']),
                           r(p['bn0_name_beta']), c_n2a_t)
    (x2_a,) = _attr_stage(False, False, (
        x1_a, a1_a[:N_ATTR], cnt_a,
        p['conv1_n2a_Wself'], p['conv1_n2a_Wneigh'], r(p['conv1_n2a_b']),
        r(p['bn1_attr_gamma']), r(p['bn1_attr_beta'])))
    pre_n2, stats2 = _name_conv(
        x1_n, c_a2n, pad_a(x1_a),
        p['conv1_a2n_Wself'], p['conv1_a2n_Wneigh'], r(p['conv1_a2n_b']))

    rows = jnp.take(pre_n2, edge_label_src, axis=0)
    out = _distmult(rows, stats2, r(p['bn1_name_gamma']),
                    r(p['bn1_name_beta']), x2_a)

    pred = jnp.tile(node_label_attr, (L,))
    return (out, edge_label_dst, pred)


# speedup vs baseline: 1.0558x; 1.0558x over previous
import functools

import jax
import jax.numpy as jnp
from jax import lax
from jax.experimental import pallas as pl
from jax.experimental.pallas import tpu as pltpu
from jax.experimental.pallas import tpu_sc as plsc

H = 256
N_NAME = 10000
N_ATTR = 1000
L = 8192
NEG = 0.01
EPS = 1e-5
NR = 10240
BN = 1024
NBLK = NR // BN

E = 160000
NS = 16
EW = E // NS
IDXR = 79
CW = 1024
CH = 1280
NCH = 8
WR = CH // NS
WEL = WR * CW
ROWPAD = NCH * CH
DUMP = CH * CW
NDUMP = 2048
CBUF = DUMP + NDUMP
CNTW = CW * NS
CNTBUF = CNTW + NDUMP
ZB = 4096


def _leaky(x):
    return jnp.where(x >= 0, x, NEG * x)


def _dot(a, b):
    return jnp.dot(a, b, preferred_element_type=jnp.float32)


def _build_counts(dst_a2n, src_a2n, src_n2a, dst_n2a):
    mesh = plsc.VectorSubcoreMesh(core_axis_name="c", subcore_axis_name="s")

    import functools as _ft

    @_ft.partial(
        pl.kernel,
        out_type=[
            jax.ShapeDtypeStruct((ROWPAD * CW,), jnp.float32),
            jax.ShapeDtypeStruct((ROWPAD * CW,), jnp.float32),
            jax.ShapeDtypeStruct((CNTW,), jnp.float32),
        ],
        mesh=mesh,
        scratch_types=[
            pltpu.VMEM((EW,), jnp.int32),
            pltpu.VMEM((EW,), jnp.int32),
            pltpu.VMEM((IDXR * 128,), jnp.int32),
            pltpu.VMEM((IDXR * 128,), jnp.float32),
            pltpu.VMEM((ZB,), jnp.float32),
            pltpu.VMEM_SHARED((CBUF,), jnp.float32),
            pltpu.VMEM_SHARED((CNTBUF,), jnp.float32),
            pltpu.SemaphoreType.DMA,
        ],
    )
    def k(d_a2n_h, s_a2n_h, s_n2a_h, d_n2a_h, out_a, out_b, out_cnt,
          rows_v, cols_v, idx1, ones2, zbuf, cbuf, cntbuf, sem):
        cid = lax.axis_index("c")
        sid = lax.axis_index("s")
        zero16 = jnp.zeros((16,), jnp.float32)

        def zinit(i, c):
            zbuf[pl.ds(i * 16, 16)] = zero16
            return c
        lax.fori_loop(0, ZB // 16, zinit, 0)
        one16 = jnp.full((16,), 1.0, jnp.float32)

        def oinit(r, c):
            ones2[pl.ds(r * 16, 16)] = one16
            return c
        lax.fori_loop(0, IDXR * 8, oinit, 0)

        wb_pending = []
        for rows_h, cols_h, out in ((d_a2n_h, s_a2n_h, out_a),
                                    (s_n2a_h, d_n2a_h, out_b)):
            with jax.named_scope("edge_stage"):
                pltpu.sync_copy(rows_h.at[pl.ds(sid * EW, EW)], rows_v)
                pltpu.sync_copy(cols_h.at[pl.ds(sid * EW, EW)], cols_v)
            for p in range(NCH // 2):
                lo = (2 * p + cid) * CH
                hi = lo + CH
                base = sid * WEL
                with jax.named_scope("zero_chunk"):
                    for h in wb_pending:
                        h.wait()
                    wb_pending = []
                    nz = WEL // ZB
                    hz = [pltpu.async_copy(
                        zbuf.at[pl.ds(0, ZB)],
                        cbuf.at[pl.ds(base + kk * ZB, ZB)], sem)
                          for kk in range(nz)]
                    tail = WEL - nz * ZB
                    if tail:
                        hz.append(pltpu.async_copy(
                            zbuf.at[pl.ds(0, tail)],
                            cbuf.at[pl.ds(base + nz * ZB, tail)], sem))

                with jax.named_scope("build_idx"):
                    iota16 = lax.iota(jnp.int32, 16)

                    def build(r, c):
                        e = r * 16
                        d = rows_v[pl.ds(e, 16)]
                        s = cols_v[pl.ds(e, 16)]
                        dmp = DUMP + (e & (NDUMP - 1)) + iota16
                        f = jnp.where((d >= lo) & (d < hi),
                                      (d - lo) * CW + s, dmp)
                        idx1[pl.ds(e, 16)] = f
                        return c
                    lax.fori_loop(0, EW // 16, build, 0)
                    for t in range(EW // 16, IDXR * 8):
                        idx1[pl.ds(t * 16, 16)] = (
                            DUMP + ((t * 16) & (NDUMP - 1)) + iota16)
                    for h in hz:
                        h.wait()

                plsc.subcore_barrier()
                with jax.named_scope("scatter"):
                    pltpu.sync_copy(ones2, cbuf.at[idx1], add=True)
                plsc.subcore_barrier()
                with jax.named_scope("writeback"):
                    wb_pending.append(pltpu.async_copy(
                        cbuf.at[pl.ds(base, WEL)],
                        out.at[pl.ds(lo * CW + base, WEL)], sem))
        for h in wb_pending:
            h.wait()

        pltpu.sync_copy(d_n2a_h.at[pl.ds(sid * EW, EW)], rows_v)
        zc = CNTBUF // NS
        pltpu.sync_copy(zbuf.at[pl.ds(0, zc)],
                        cntbuf.at[pl.ds(sid * zc, zc)])
        iota16c = lax.iota(jnp.int32, 16)

        def build_cnt(r, c):
            e = r * 16
            idx1[pl.ds(e, 16)] = rows_v[pl.ds(e, 16)] * NS + sid
            return c
        lax.fori_loop(0, EW // 16, build_cnt, 0)
        for t in range(EW // 16, IDXR * 8):
            idx1[pl.ds(t * 16, 16)] = (
                CNTW + ((t * 16) & (NDUMP - 1)) + iota16c)

        plsc.subcore_barrier()
        with jax.named_scope("scatter_cnt"):
            pltpu.sync_copy(ones2, cntbuf.at[idx1], add=True)
        plsc.subcore_barrier()

        @pl.when(jnp.logical_and(cid == 0, sid == 0))
        def _():
            pltpu.sync_copy(cntbuf.at[pl.ds(0, CNTW)], out_cnt)

    return k(dst_a2n, src_a2n, src_n2a, dst_n2a)


def _tdot(ct, h):
    return lax.dot_general(ct, h, (((0,), (0,)), ((), ())),
                           preferred_element_type=jnp.float32)


def _enc_name_body(g, w0, b0, w1, b1, ct, x_out, a_out, acc_a):
    i = pl.program_id(0)
    h = _leaky(_dot(g[...], w0[...]) + b0[...])
    h = _leaky(_dot(h, w1[...]) + b1[...])
    x_out[...] = h

    @pl.when(i == 0)
    def _():
        acc_a[...] = jnp.zeros_like(acc_a)

    acc_a[...] += _tdot(ct[...].astype(jnp.float32), h)

    @pl.when(i == NBLK - 1)
    def _():
        a_out[...] = acc_a[...]


def _enc_name(g_n, w0, b0, w1, b1, c_n2a_t):
    return pl.pallas_call(
        _enc_name_body,
        grid=(NBLK,),
        in_specs=[
            pl.BlockSpec((BN, H), lambda i: (i, 0)),
            pl.BlockSpec((H, H), lambda i: (0, 0)),
            pl.BlockSpec((1, H), lambda i: (0, 0)),
            pl.BlockSpec((H, H), lambda i: (0, 0)),
            pl.BlockSpec((1, H), lambda i: (0, 0)),
            pl.BlockSpec((BN, CW), lambda i: (i, 0)),
        ],
        out_specs=[
            pl.BlockSpec((BN, H), lambda i: (i, 0)),
            pl.BlockSpec((CW, H), lambda i: (0, 0)),
        ],
        out_shape=[
            jax.ShapeDtypeStruct((NR, H), jnp.float32),
            jax.ShapeDtypeStruct((CW, H), jnp.float32),
        ],
        scratch_shapes=[
            pltpu.VMEM((CW, H), jnp.float32),
        ],
    )(g_n, w0, b0, w1, b1, c_n2a_t)


def _attr_stage_body(with_enc, with_leaky, *refs):
    if with_enc:
        (g, w0, b0, w1, b1, agg, cnt, ws, wn, bb, gamma, beta, x_enc_out,
         x_out) = refs
        h = _leaky(_dot(g[...], w0[...]) + b0[...])
        h = _leaky(_dot(h, w1[...]) + b1[...])
        x_enc_out[...] = h
    else:
        (g, agg, cnt, ws, wn, bb, gamma, beta, x_out) = refs
        h = g[...]
    cn = jnp.sum(cnt[...], axis=1, keepdims=True)[:N_ATTR]
    aggr = agg[...] / jnp.maximum(cn, 1.0)
    pre = _dot(h, ws[...]) + _dot(aggr, wn[...]) + bb[...]
    mu = jnp.mean(pre, axis=0, keepdims=True)
    var = jnp.mean((pre - mu) ** 2, axis=0, keepdims=True)
    y = (pre - mu) * lax.rsqrt(var + EPS) * gamma[...] + beta[...]
    if with_leaky:
        y = _leaky(y)
    x_out[...] = y


def _attr_stage(with_enc, with_leaky, args):
    n_in = len(args)
    n_out = 2 if with_enc else 1
    full = lambda s: pl.BlockSpec(s, lambda: (0, 0))
    in_specs = [full(a.shape) for a in args]
    return pl.pallas_call(
        functools.partial(_attr_stage_body, with_enc, with_leaky),
        grid=(),
        in_specs=in_specs,
        out_specs=[full((N_ATTR, H))] * n_out,
        out_shape=[jax.ShapeDtypeStruct((N_ATTR, H), jnp.float32)] * n_out,
    )(*args)


def _name_conv_body(x, c, xa, ws, wn, bb, pre_out, stats_out, s1, s2):
    i = pl.program_id(0)
    cb = c[...].astype(jnp.float32)
    rs = jnp.sum(cb, axis=1, keepdims=True)
    aggr = _dot(cb, xa[...]) / jnp.maximum(rs, 1.0)
    pre = _dot(x[...], ws[...]) + _dot(aggr, wn[...]) + bb[...]
    pre_out[...] = pre

    @pl.when(i == 0)
    def _():
        s1[...] = jnp.zeros_like(s1)
        s2[...] = jnp.zeros_like(s2)

    row = i * BN + lax.broadcasted_iota(jnp.int32, (BN, 1), 0)
    pm = jnp.where(row < N_NAME, pre, 0.0)
    s1[...] += jnp.sum(pm, axis=0, keepdims=True)
    s2[...] += jnp.sum(pm * pm, axis=0, keepdims=True)

    @pl.when(i == NBLK - 1)
    def _():
        stats_out[0:1, :] = s1[...]
        stats_out[1:2, :] = s2[...]


def _name_conv(x_n, c_a2n, x_a, ws, wn, bb):
    return pl.pallas_call(
        _name_conv_body,
        grid=(NBLK,),
        in_specs=[
            pl.BlockSpec((BN, H), lambda i: (i, 0)),
            pl.BlockSpec((BN, CW), lambda i: (i, 0)),
            pl.BlockSpec((CW, H), lambda i: (0, 0)),
            pl.BlockSpec((H, H), lambda i: (0, 0)),
            pl.BlockSpec((H, H), lambda i: (0, 0)),
            pl.BlockSpec((1, H), lambda i: (0, 0)),
        ],
        out_specs=[
            pl.BlockSpec((BN, H), lambda i: (i, 0)),
            pl.BlockSpec((2, H), lambda i: (0, 0)),
        ],
        out_shape=[
            jax.ShapeDtypeStruct((NR, H), jnp.float32),
            jax.ShapeDtypeStruct((2, H), jnp.float32),
        ],
        scratch_shapes=[
            pltpu.VMEM((1, H), jnp.float32),
            pltpu.VMEM((1, H), jnp.float32),
        ],
    )(x_n, c_a2n, x_a, ws, wn, bb)


def _bn_accum_body(pre, stats, gamma, beta, c, x_out, a_out, acc):
    i = pl.program_id(0)
    mu = stats[0:1, :] / N_NAME
    var = stats[1:2, :] / N_NAME - mu * mu
    y = (pre[...] - mu) * lax.rsqrt(var + EPS) * gamma[...] + beta[...]
    y = _leaky(y)
    x_out[...] = y

    @pl.when(i == 0)
    def _():
        acc[...] = jnp.zeros_like(acc)

    acc[...] += _tdot(c[...].astype(jnp.float32), y)

    @pl.when(i == NBLK - 1)
    def _():
        a_out[...] = acc[...]


def _bn_accum(pre_n, stats, gamma, beta, c_n2a_t):
    return pl.pallas_call(
        _bn_accum_body,
        grid=(NBLK,),
        in_specs=[
            pl.BlockSpec((BN, H), lambda i: (i, 0)),
            pl.BlockSpec((2, H), lambda i: (0, 0)),
            pl.BlockSpec((1, H), lambda i: (0, 0)),
            pl.BlockSpec((1, H), lambda i: (0, 0)),
            pl.BlockSpec((BN, CW), lambda i: (i, 0)),
        ],
        out_specs=[
            pl.BlockSpec((BN, H), lambda i: (i, 0)),
            pl.BlockSpec((CW, H), lambda i: (0, 0)),
        ],
        out_shape=[
            jax.ShapeDtypeStruct((NR, H), jnp.float32),
            jax.ShapeDtypeStruct((CW, H), jnp.float32),
        ],
        scratch_shapes=[pltpu.VMEM((CW, H), jnp.float32)],
    )(pre_n, stats, gamma, beta, c_n2a_t)


LB = 1024
LBLK = L // LB


def _distmult_body(rows, stats, gamma, beta, xa, out):
    mu = stats[0:1, :] / N_NAME
    var = stats[1:2, :] / N_NAME - mu * mu
    y = (rows[...] - mu) * lax.rsqrt(var + EPS) * gamma[...] + beta[...]
    out[...] = lax.dot_general(y, xa[...], (((1,), (1,)), ((), ())),
                               preferred_element_type=jnp.float32)


def _distmult(rows, stats, gamma, beta, x_a):
    return pl.pallas_call(
        _distmult_body,
        grid=(LBLK,),
        in_specs=[
            pl.BlockSpec((LB, H), lambda i: (i, 0)),
            pl.BlockSpec((2, H), lambda i: (0, 0)),
            pl.BlockSpec((1, H), lambda i: (0, 0)),
            pl.BlockSpec((1, H), lambda i: (0, 0)),
            pl.BlockSpec((N_ATTR, H), lambda i: (0, 0)),
        ],
        out_specs=pl.BlockSpec((LB, N_ATTR), lambda i: (i, 0)),
        out_shape=jax.ShapeDtypeStruct((L, N_ATTR), jnp.float32),
    )(rows, stats, gamma, beta, x_a)


def kernel(params, node_feature_name, node_feature_attr, edge_src_n2a,
           edge_dst_n2a, edge_src_a2n, edge_dst_a2n, edge_label_src,
           edge_label_dst, node_label_attr):
    p = params
    r = lambda v: jnp.reshape(v, (1, H))

    idx_n = jnp.concatenate(
        [node_feature_name[:, 0], jnp.zeros((NR - N_NAME,), jnp.int32)])
    g_n = jnp.take(p['emb_name'], idx_n, axis=0)
    g_a = jnp.take(p['emb_attr'], node_feature_attr[:, 0], axis=0)
    ca_flat, cbt_flat, cnt_raw = _build_counts(
        edge_dst_a2n, edge_src_a2n, edge_src_n2a, edge_dst_n2a)
    c_a2n = jnp.reshape(ca_flat, (ROWPAD, CW))
    c_n2a_t = jnp.reshape(cbt_flat, (ROWPAD, CW))
    cnt_a = jnp.reshape(cnt_raw, (CW, NS))
    pad_a = lambda v: jnp.pad(v, ((0, CW - N_ATTR), (0, 0)))

    x0_n, a0_a = _enc_name(
        g_n, p['mlp_name_W0'], r(p['mlp_name_b0']),
        p['mlp_name_W1'], r(p['mlp_name_b1']), c_n2a_t)
    x0_a, x1_a = _attr_stage(True, True, (
        g_a, p['mlp_attr_W0'], r(p['mlp_attr_b0']),
        p['mlp_attr_W1'], r(p['mlp_attr_b1']),
        a0_a[:N_ATTR], cnt_a,
        p['conv0_n2a_Wself'], p['conv0_n2a_Wneigh'], r(p['conv0_n2a_b']),
        r(p['bn0_attr_gamma']), r(p['bn0_attr_beta'])))
    pre_n1, stats1 = _name_conv(
        x0_n, c_a2n, pad_a(x0_a),
        p['conv0_a2n_Wself'], p['conv0_a2n_Wneigh'], r(p['conv0_a2n_b']))
    x1_n, a1_a = _bn_accum(pre_n1, stats1, r(p['bn0_name_gamma']),
                           r(p['bn0_name_beta']), c_n2a_t)
    (x2_a,) = _attr_stage(False, False, (
        x1_a, a1_a[:N_ATTR], cnt_a,
        p['conv1_n2a_Wself'], p['conv1_n2a_Wneigh'], r(p['conv1_n2a_b']),
        r(p['bn1_attr_gamma']), r(p['bn1_attr_beta'])))
    pre_n2, stats2 = _name_conv(
        x1_n, c_a2n, pad_a(x1_a),
        p['conv1_a2n_Wself'], p['conv1_a2n_Wneigh'], r(p['conv1_a2n_b']))

    rows = jnp.take(pre_n2, edge_label_src, axis=0)
    out = _distmult(rows, stats2, r(p['bn1_name_gamma']),
                    r(p['bn1_name_beta']), x2_a)

    pred = jnp.tile(node_label_attr, (L,))
    return (out, edge_label_dst, pred)


# speedup vs baseline: 1.2714x; 1.2042x over previous
import functools

import jax
import jax.numpy as jnp
from jax import lax
from jax.experimental import pallas as pl
from jax.experimental.pallas import tpu as pltpu
from jax.experimental.pallas import tpu_sc as plsc

H = 256
N_NAME = 10000
N_ATTR = 1000
L = 8192
NEG = 0.01
EPS = 1e-5
NR = 10240
BN = 1280
NBLK = NR // BN

E = 160000
NS = 16
EW = E // NS
IDXR = 79
CW = 1024
CH = 1280
NCH = 8
WR = CH // NS
WEL = WR * CW
ROWPAD = NCH * CH
DUMP = CH * CW
NDUMP = 2048
CBUF = DUMP + NDUMP
CNTW = CW * NS
CNTBUF = CNTW + NDUMP
ZB = 4096


def _leaky(x):
    return jnp.where(x >= 0, x, NEG * x)


def _dot(a, b):
    return jnp.dot(a, b, preferred_element_type=jnp.float32)


def _build_counts(dst_a2n, src_a2n, src_n2a, dst_n2a):
    mesh = plsc.VectorSubcoreMesh(core_axis_name="c", subcore_axis_name="s")

    import functools as _ft

    @_ft.partial(
        pl.kernel,
        out_type=[
            jax.ShapeDtypeStruct((ROWPAD * CW,), jnp.float32),
            jax.ShapeDtypeStruct((ROWPAD * CW,), jnp.float32),
            jax.ShapeDtypeStruct((CNTW,), jnp.float32),
        ],
        mesh=mesh,
        scratch_types=[
            pltpu.VMEM((EW,), jnp.int32),
            pltpu.VMEM((EW,), jnp.int32),
            pltpu.VMEM((IDXR * 128,), jnp.int32),
            pltpu.VMEM((IDXR * 128,), jnp.float32),
            pltpu.VMEM((ZB,), jnp.float32),
            pltpu.VMEM_SHARED((CBUF,), jnp.float32),
            pltpu.VMEM_SHARED((CNTBUF,), jnp.float32),
            pltpu.SemaphoreType.DMA,
        ],
    )
    def k(d_a2n_h, s_a2n_h, s_n2a_h, d_n2a_h, out_a, out_b, out_cnt,
          rows_v, cols_v, idx1, ones2, zbuf, cbuf, cntbuf, sem):
        cid = lax.axis_index("c")
        sid = lax.axis_index("s")
        zero16 = jnp.zeros((16,), jnp.float32)

        def zinit(i, c):
            zbuf[pl.ds(i * 16, 16)] = zero16
            return c
        lax.fori_loop(0, ZB // 16, zinit, 0)
        one16 = jnp.full((16,), 1.0, jnp.float32)

        def oinit(r, c):
            ones2[pl.ds(r * 16, 16)] = one16
            return c
        lax.fori_loop(0, IDXR * 8, oinit, 0)

        wb_pending = []
        for rows_h, cols_h, out in ((d_a2n_h, s_a2n_h, out_a),
                                    (s_n2a_h, d_n2a_h, out_b)):
            with jax.named_scope("edge_stage"):
                pltpu.sync_copy(rows_h.at[pl.ds(sid * EW, EW)], rows_v)
                pltpu.sync_copy(cols_h.at[pl.ds(sid * EW, EW)], cols_v)
            for p in range(NCH // 2):
                lo = (2 * p + cid) * CH
                hi = lo + CH
                base = sid * WEL
                with jax.named_scope("zero_chunk"):
                    for h in wb_pending:
                        h.wait()
                    wb_pending = []
                    nz = WEL // ZB
                    hz = [pltpu.async_copy(
                        zbuf.at[pl.ds(0, ZB)],
                        cbuf.at[pl.ds(base + kk * ZB, ZB)], sem)
                          for kk in range(nz)]
                    tail = WEL - nz * ZB
                    if tail:
                        hz.append(pltpu.async_copy(
                            zbuf.at[pl.ds(0, tail)],
                            cbuf.at[pl.ds(base + nz * ZB, tail)], sem))

                with jax.named_scope("build_idx"):
                    iota16 = lax.iota(jnp.int32, 16)

                    def build(r, c):
                        e = r * 16
                        d = rows_v[pl.ds(e, 16)]
                        s = cols_v[pl.ds(e, 16)]
                        dmp = DUMP + (e & (NDUMP - 1)) + iota16
                        f = ((s >> 7) * (CH * 128) + (d - lo) * 128
                             + (s & 127))
                        f = jnp.where((d >= lo) & (d < hi), f, dmp)
                        idx1[pl.ds(e, 16)] = f
                        return c
                    lax.fori_loop(0, EW // 16, build, 0)
                    for t in range(EW // 16, IDXR * 8):
                        idx1[pl.ds(t * 16, 16)] = (
                            DUMP + ((t * 16) & (NDUMP - 1)) + iota16)
                    for h in hz:
                        h.wait()

                plsc.subcore_barrier()
                with jax.named_scope("scatter"):
                    pltpu.sync_copy(ones2, cbuf.at[idx1], add=True)
                plsc.subcore_barrier()
                with jax.named_scope("writeback"):
                    wb_pending.append(pltpu.async_copy(
                        cbuf.at[pl.ds(base, WEL)],
                        out.at[pl.ds(lo * CW + base, WEL)], sem))
        for h in wb_pending:
            h.wait()

        pltpu.sync_copy(d_n2a_h.at[pl.ds(sid * EW, EW)], rows_v)
        zc = CNTBUF // NS
        pltpu.sync_copy(zbuf.at[pl.ds(0, zc)],
                        cntbuf.at[pl.ds(sid * zc, zc)])
        iota16c = lax.iota(jnp.int32, 16)

        def build_cnt(r, c):
            e = r * 16
            idx1[pl.ds(e, 16)] = rows_v[pl.ds(e, 16)] * NS + sid
            return c
        lax.fori_loop(0, EW // 16, build_cnt, 0)
        for t in range(EW // 16, IDXR * 8):
            idx1[pl.ds(t * 16, 16)] = (
                CNTW + ((t * 16) & (NDUMP - 1)) + iota16c)

        plsc.subcore_barrier()
        with jax.named_scope("scatter_cnt"):
            pltpu.sync_copy(ones2, cntbuf.at[idx1], add=True)
        plsc.subcore_barrier()

        @pl.when(jnp.logical_and(cid == 0, sid == 0))
        def _():
            pltpu.sync_copy(cntbuf.at[pl.ds(0, CNTW)], out_cnt)

    return k(dst_a2n, src_a2n, src_n2a, dst_n2a)


def _tdot(ct, h):
    return lax.dot_general(ct, h, (((0,), (0,)), ((), ())),
                           preferred_element_type=jnp.float32)


def _enc_name_body(g, w0, b0, w1, b1, ct, x_out, a_out, acc_a):
    i = pl.program_id(0)
    h = _leaky(_dot(g[...], w0[...]) + b0[...])
    h = _leaky(_dot(h, w1[...]) + b1[...])
    x_out[...] = h

    @pl.when(i == 0)
    def _():
        acc_a[...] = jnp.zeros_like(acc_a)

    for k in range(CW // 128):
        acc_a[pl.ds(k * 128, 128), :] += _tdot(ct[0, k], h)

    @pl.when(i == NBLK - 1)
    def _():
        a_out[...] = acc_a[...]


def _enc_name(g_n, w0, b0, w1, b1, c_n2a_t):
    return pl.pallas_call(
        _enc_name_body,
        grid=(NBLK,),
        in_specs=[
            pl.BlockSpec((BN, H), lambda i: (i, 0)),
            pl.BlockSpec((H, H), lambda i: (0, 0)),
            pl.BlockSpec((1, H), lambda i: (0, 0)),
            pl.BlockSpec((H, H), lambda i: (0, 0)),
            pl.BlockSpec((1, H), lambda i: (0, 0)),
            pl.BlockSpec((1, CW // 128, CH, 128), lambda i: (i, 0, 0, 0)),
        ],
        out_specs=[
            pl.BlockSpec((BN, H), lambda i: (i, 0)),
            pl.BlockSpec((CW, H), lambda i: (0, 0)),
        ],
        out_shape=[
            jax.ShapeDtypeStruct((NR, H), jnp.float32),
            jax.ShapeDtypeStruct((CW, H), jnp.float32),
        ],
        scratch_shapes=[
            pltpu.VMEM((CW, H), jnp.float32),
        ],
    )(g_n, w0, b0, w1, b1, c_n2a_t)


def _attr_stage_body(with_enc, with_leaky, *refs):
    if with_enc:
        (g, w0, b0, w1, b1, agg, cnt, ws, wn, bb, gamma, beta, x_enc_out,
         x_out) = refs
        h = _leaky(_dot(g[...], w0[...]) + b0[...])
        h = _leaky(_dot(h, w1[...]) + b1[...])
        x_enc_out[...] = h
    else:
        (g, agg, cnt, ws, wn, bb, gamma, beta, x_out) = refs
        h = g[...]
    cn = jnp.sum(cnt[...], axis=1, keepdims=True)[:N_ATTR]
    aggr = agg[...] / jnp.maximum(cn, 1.0)
    pre = _dot(h, ws[...]) + _dot(aggr, wn[...]) + bb[...]
    mu = jnp.mean(pre, axis=0, keepdims=True)
    var = jnp.mean((pre - mu) ** 2, axis=0, keepdims=True)
    y = (pre - mu) * lax.rsqrt(var + EPS) * gamma[...] + beta[...]
    if with_leaky:
        y = _leaky(y)
    x_out[...] = y


def _attr_stage(with_enc, with_leaky, args):
    n_in = len(args)
    n_out = 2 if with_enc else 1
    full = lambda s: pl.BlockSpec(s, lambda: (0, 0))
    in_specs = [full(a.shape) for a in args]
    return pl.pallas_call(
        functools.partial(_attr_stage_body, with_enc, with_leaky),
        grid=(),
        in_specs=in_specs,
        out_specs=[full((N_ATTR, H))] * n_out,
        out_shape=[jax.ShapeDtypeStruct((N_ATTR, H), jnp.float32)] * n_out,
    )(*args)


def _name_conv_body(x, c, xa, ws, wn, bb, pre_out, stats_out, s1, s2):
    i = pl.program_id(0)
    aggr = jnp.zeros((BN, H), jnp.float32)
    rs = jnp.zeros((BN, 1), jnp.float32)
    for k in range(CW // 128):
        ck = c[0, k]
        aggr += _dot(ck, xa[pl.ds(k * 128, 128), :])
        rs += jnp.sum(ck, axis=1, keepdims=True)
    aggr = aggr / jnp.maximum(rs, 1.0)
    pre = _dot(x[...], ws[...]) + _dot(aggr, wn[...]) + bb[...]
    pre_out[...] = pre

    @pl.when(i == 0)
    def _():
        s1[...] = jnp.zeros_like(s1)
        s2[...] = jnp.zeros_like(s2)

    row = i * BN + lax.broadcasted_iota(jnp.int32, (BN, 1), 0)
    pm = jnp.where(row < N_NAME, pre, 0.0)
    s1[...] += jnp.sum(pm, axis=0, keepdims=True)
    s2[...] += jnp.sum(pm * pm, axis=0, keepdims=True)

    @pl.when(i == NBLK - 1)
    def _():
        stats_out[0:1, :] = s1[...]
        stats_out[1:2, :] = s2[...]


def _name_conv(x_n, c_a2n, x_a, ws, wn, bb):
    return pl.pallas_call(
        _name_conv_body,
        grid=(NBLK,),
        in_specs=[
            pl.BlockSpec((BN, H), lambda i: (i, 0)),
            pl.BlockSpec((1, CW // 128, CH, 128), lambda i: (i, 0, 0, 0)),
            pl.BlockSpec((CW, H), lambda i: (0, 0)),
            pl.BlockSpec((H, H), lambda i: (0, 0)),
            pl.BlockSpec((H, H), lambda i: (0, 0)),
            pl.BlockSpec((1, H), lambda i: (0, 0)),
        ],
        out_specs=[
            pl.BlockSpec((BN, H), lambda i: (i, 0)),
            pl.BlockSpec((2, H), lambda i: (0, 0)),
        ],
        out_shape=[
            jax.ShapeDtypeStruct((NR, H), jnp.float32),
            jax.ShapeDtypeStruct((2, H), jnp.float32),
        ],
        scratch_shapes=[
            pltpu.VMEM((1, H), jnp.float32),
            pltpu.VMEM((1, H), jnp.float32),
        ],
    )(x_n, c_a2n, x_a, ws, wn, bb)


def _bn_accum_body(pre, stats, gamma, beta, c, x_out, a_out, acc):
    i = pl.program_id(0)
    mu = stats[0:1, :] / N_NAME
    var = stats[1:2, :] / N_NAME - mu * mu
    y = (pre[...] - mu) * lax.rsqrt(var + EPS) * gamma[...] + beta[...]
    y = _leaky(y)
    x_out[...] = y

    @pl.when(i == 0)
    def _():
        acc[...] = jnp.zeros_like(acc)

    for k in range(CW // 128):
        acc[pl.ds(k * 128, 128), :] += _tdot(c[0, k], y)

    @pl.when(i == NBLK - 1)
    def _():
        a_out[...] = acc[...]


def _bn_accum(pre_n, stats, gamma, beta, c_n2a_t):
    return pl.pallas_call(
        _bn_accum_body,
        grid=(NBLK,),
        in_specs=[
            pl.BlockSpec((BN, H), lambda i: (i, 0)),
            pl.BlockSpec((2, H), lambda i: (0, 0)),
            pl.BlockSpec((1, H), lambda i: (0, 0)),
            pl.BlockSpec((1, H), lambda i: (0, 0)),
            pl.BlockSpec((1, CW // 128, CH, 128), lambda i: (i, 0, 0, 0)),
        ],
        out_specs=[
            pl.BlockSpec((BN, H), lambda i: (i, 0)),
            pl.BlockSpec((CW, H), lambda i: (0, 0)),
        ],
        out_shape=[
            jax.ShapeDtypeStruct((NR, H), jnp.float32),
            jax.ShapeDtypeStruct((CW, H), jnp.float32),
        ],
        scratch_shapes=[pltpu.VMEM((CW, H), jnp.float32)],
    )(pre_n, stats, gamma, beta, c_n2a_t)


LB = 1024
LBLK = L // LB


def _distmult_body(rows, stats, gamma, beta, xa, out):
    mu = stats[0:1, :] / N_NAME
    var = stats[1:2, :] / N_NAME - mu * mu
    y = (rows[...] - mu) * lax.rsqrt(var + EPS) * gamma[...] + beta[...]
    out[...] = lax.dot_general(y, xa[...], (((1,), (1,)), ((), ())),
                               preferred_element_type=jnp.float32)


def _distmult(rows, stats, gamma, beta, x_a):
    return pl.pallas_call(
        _distmult_body,
        grid=(LBLK,),
        in_specs=[
            pl.BlockSpec((LB, H), lambda i: (i, 0)),
            pl.BlockSpec((2, H), lambda i: (0, 0)),
            pl.BlockSpec((1, H), lambda i: (0, 0)),
            pl.BlockSpec((1, H), lambda i: (0, 0)),
            pl.BlockSpec((N_ATTR, H), lambda i: (0, 0)),
        ],
        out_specs=pl.BlockSpec((LB, N_ATTR), lambda i: (i, 0)),
        out_shape=jax.ShapeDtypeStruct((L, N_ATTR), jnp.float32),
    )(rows, stats, gamma, beta, x_a)


def kernel(params, node_feature_name, node_feature_attr, edge_src_n2a,
           edge_dst_n2a, edge_src_a2n, edge_dst_a2n, edge_label_src,
           edge_label_dst, node_label_attr):
    p = params
    r = lambda v: jnp.reshape(v, (1, H))

    idx_n = jnp.concatenate(
        [node_feature_name[:, 0], jnp.zeros((NR - N_NAME,), jnp.int32)])
    g_n = jnp.take(p['emb_name'], idx_n, axis=0)
    g_a = jnp.take(p['emb_attr'], node_feature_attr[:, 0], axis=0)
    ca_flat, cbt_flat, cnt_raw = _build_counts(
        edge_dst_a2n, edge_src_a2n, edge_src_n2a, edge_dst_n2a)
    c_a2n = jnp.reshape(ca_flat, (NCH, CW // 128, CH, 128))
    c_n2a_t = jnp.reshape(cbt_flat, (NCH, CW // 128, CH, 128))
    cnt_a = jnp.reshape(cnt_raw, (CW, NS))
    pad_a = lambda v: jnp.pad(v, ((0, CW - N_ATTR), (0, 0)))

    x0_n, a0_a = _enc_name(
        g_n, p['mlp_name_W0'], r(p['mlp_name_b0']),
        p['mlp_name_W1'], r(p['mlp_name_b1']), c_n2a_t)
    x0_a, x1_a = _attr_stage(True, True, (
        g_a, p['mlp_attr_W0'], r(p['mlp_attr_b0']),
        p['mlp_attr_W1'], r(p['mlp_attr_b1']),
        a0_a[:N_ATTR], cnt_a,
        p['conv0_n2a_Wself'], p['conv0_n2a_Wneigh'], r(p['conv0_n2a_b']),
        r(p['bn0_attr_gamma']), r(p['bn0_attr_beta'])))
    pre_n1, stats1 = _name_conv(
        x0_n, c_a2n, pad_a(x0_a),
        p['conv0_a2n_Wself'], p['conv0_a2n_Wneigh'], r(p['conv0_a2n_b']))
    x1_n, a1_a = _bn_accum(pre_n1, stats1, r(p['bn0_name_gamma']),
                           r(p['bn0_name_beta']), c_n2a_t)
    (x2_a,) = _attr_stage(False, False, (
        x1_a, a1_a[:N_ATTR], cnt_a,
        p['conv1_n2a_Wself'], p['conv1_n2a_Wneigh'], r(p['conv1_n2a_b']),
        r(p['bn1_attr_gamma']), r(p['bn1_attr_beta'])))
    pre_n2, stats2 = _name_conv(
        x1_n, c_a2n, pad_a(x1_a),
        p['conv1_a2n_Wself'], p['conv1_a2n_Wneigh'], r(p['conv1_a2n_b']))

    rows = jnp.take(pre_n2, edge_label_src, axis=0)
    out = _distmult(rows, stats2, r(p['bn1_name_gamma']),
                    r(p['bn1_name_beta']), x2_a)

    pred = jnp.tile(node_label_attr, (L,))
    return (out, edge_label_dst, pred)


# speedup vs baseline: 1.4151x; 1.1130x over previous
import functools

import jax
import jax.numpy as jnp
from jax import lax
from jax.experimental import pallas as pl
from jax.experimental.pallas import tpu as pltpu
from jax.experimental.pallas import tpu_sc as plsc

H = 256
N_NAME = 10000
N_ATTR = 1000
L = 8192
NEG = 0.01
EPS = 1e-5
NR = 10240
BN = 1280
NBLK = NR // BN

E = 160000
NS = 16
EW = E // NS
IDXR = 79
CW = 1024
CH = 1280
NCH = 8
WR = CH // NS
WEL = WR * CW
ROWPAD = NCH * CH
DUMP = CH * CW
NDUMP = 2048
CBUF = DUMP + NDUMP
CNTW = CW * NS
CNTBUF = CNTW + NDUMP
ZB = 4096


def _leaky(x):
    return jnp.where(x >= 0, x, NEG * x)


def _dot(a, b):
    return jnp.dot(a, b, preferred_element_type=jnp.float32)


def _build_counts(dst_a2n, src_a2n, src_n2a, dst_n2a):
    mesh = plsc.VectorSubcoreMesh(core_axis_name="c", subcore_axis_name="s")

    import functools as _ft

    @_ft.partial(
        pl.kernel,
        out_type=[
            jax.ShapeDtypeStruct((ROWPAD * CW,), jnp.float32),
            jax.ShapeDtypeStruct((ROWPAD * CW,), jnp.float32),
            jax.ShapeDtypeStruct((CNTW,), jnp.float32),
        ],
        mesh=mesh,
        scratch_types=[
            pltpu.VMEM((EW,), jnp.int32),
            pltpu.VMEM((EW,), jnp.int32),
            pltpu.VMEM((IDXR * 128,), jnp.int32),
            pltpu.VMEM((IDXR * 128,), jnp.float32),
            pltpu.VMEM((ZB,), jnp.float32),
            pltpu.VMEM_SHARED((CBUF,), jnp.float32),
            pltpu.VMEM_SHARED((CNTBUF,), jnp.float32),
            pltpu.SemaphoreType.DMA,
        ],
    )
    def k(d_a2n_h, s_a2n_h, s_n2a_h, d_n2a_h, out_a, out_b, out_cnt,
          rows_v, cols_v, idx1, ones2, zbuf, cbuf, cntbuf, sem):
        cid = lax.axis_index("c")
        sid = lax.axis_index("s")
        zero16 = jnp.zeros((16,), jnp.float32)

        def zinit(i, c):
            zbuf[pl.ds(i * 16, 16)] = zero16
            return c
        lax.fori_loop(0, ZB // 16, zinit, 0)
        one16 = jnp.full((16,), 1.0, jnp.float32)

        def oinit(r, c):
            ones2[pl.ds(r * 16, 16)] = one16
            return c
        lax.fori_loop(0, IDXR * 8, oinit, 0)

        wb_pending = []
        for rows_h, cols_h, out in ((d_a2n_h, s_a2n_h, out_a),
                                    (s_n2a_h, d_n2a_h, out_b)):
            with jax.named_scope("edge_stage"):
                pltpu.sync_copy(rows_h.at[pl.ds(sid * EW, EW)], rows_v)
                pltpu.sync_copy(cols_h.at[pl.ds(sid * EW, EW)], cols_v)
            for p in range(NCH // 2):
                lo = (2 * p + cid) * CH
                hi = lo + CH
                base = sid * WEL
                with jax.named_scope("zero_chunk"):
                    for h in wb_pending:
                        h.wait()
                    wb_pending = []
                    nz = WEL // ZB
                    hz = [pltpu.async_copy(
                        zbuf.at[pl.ds(0, ZB)],
                        cbuf.at[pl.ds(base + kk * ZB, ZB)], sem)
                          for kk in range(nz)]
                    tail = WEL - nz * ZB
                    if tail:
                        hz.append(pltpu.async_copy(
                            zbuf.at[pl.ds(0, tail)],
                            cbuf.at[pl.ds(base + nz * ZB, tail)], sem))

                with jax.named_scope("build_idx"):
                    iota16 = lax.iota(jnp.int32, 16)

                    def build(r, c):
                        e = r * 16
                        d = rows_v[pl.ds(e, 16)]
                        s = cols_v[pl.ds(e, 16)]
                        dmp = DUMP + (e & (NDUMP - 1)) + iota16
                        f = ((s >> 7) * (CH * 128) + (d - lo) * 128
                             + (s & 127))
                        f = jnp.where((d >= lo) & (d < hi), f, dmp)
                        idx1[pl.ds(e, 16)] = f
                        return c
                    lax.fori_loop(0, EW // 16, build, 0)
                    for t in range(EW // 16, IDXR * 8):
                        idx1[pl.ds(t * 16, 16)] = (
                            DUMP + ((t * 16) & (NDUMP - 1)) + iota16)
                    for h in hz:
                        h.wait()

                plsc.subcore_barrier()
                with jax.named_scope("scatter"):
                    pltpu.sync_copy(ones2, cbuf.at[idx1], add=True)
                plsc.subcore_barrier()
                with jax.named_scope("writeback"):
                    wb_pending.append(pltpu.async_copy(
                        cbuf.at[pl.ds(base, WEL)],
                        out.at[pl.ds(lo * CW + base, WEL)], sem))
        for h in wb_pending:
            h.wait()

        pltpu.sync_copy(d_n2a_h.at[pl.ds(sid * EW, EW)], rows_v)
        zc = CNTBUF // NS
        pltpu.sync_copy(zbuf.at[pl.ds(0, zc)],
                        cntbuf.at[pl.ds(sid * zc, zc)])
        iota16c = lax.iota(jnp.int32, 16)

        def build_cnt(r, c):
            e = r * 16
            idx1[pl.ds(e, 16)] = rows_v[pl.ds(e, 16)] * NS + sid
            return c
        lax.fori_loop(0, EW // 16, build_cnt, 0)
        for t in range(EW // 16, IDXR * 8):
            idx1[pl.ds(t * 16, 16)] = (
                CNTW + ((t * 16) & (NDUMP - 1)) + iota16c)

        plsc.subcore_barrier()
        with jax.named_scope("scatter_cnt"):
            pltpu.sync_copy(ones2, cntbuf.at[idx1], add=True)
        plsc.subcore_barrier()

        @pl.when(jnp.logical_and(cid == 0, sid == 0))
        def _():
            pltpu.sync_copy(cntbuf.at[pl.ds(0, CNTW)], out_cnt)

    return k(dst_a2n, src_a2n, src_n2a, dst_n2a)


def _tdot(ct, h):
    return lax.dot_general(ct, h, (((0,), (0,)), ((), ())),
                           preferred_element_type=jnp.float32)


def _enc_name_body(g, w0, b0, w1, b1, ct, x_out, a_out, acc_a):
    i = pl.program_id(0)
    h = _leaky(_dot(g[...], w0[...]) + b0[...])
    h = _leaky(_dot(h, w1[...]) + b1[...])
    x_out[...] = h

    @pl.when(i == 0)
    def _():
        acc_a[...] = jnp.zeros_like(acc_a)

    for k in range(CW // 128):
        acc_a[pl.ds(k * 128, 128), :] += _tdot(ct[0, k], h)

    @pl.when(i == NBLK - 1)
    def _():
        a_out[...] = acc_a[...]


def _enc_name(g_n, w0, b0, w1, b1, c_n2a_t):
    return pl.pallas_call(
        _enc_name_body,
        grid=(NBLK,),
        in_specs=[
            pl.BlockSpec((BN, H), lambda i: (i, 0)),
            pl.BlockSpec((H, H), lambda i: (0, 0)),
            pl.BlockSpec((1, H), lambda i: (0, 0)),
            pl.BlockSpec((H, H), lambda i: (0, 0)),
            pl.BlockSpec((1, H), lambda i: (0, 0)),
            pl.BlockSpec((1, CW // 128, CH, 128), lambda i: (i, 0, 0, 0)),
        ],
        out_specs=[
            pl.BlockSpec((BN, H), lambda i: (i, 0)),
            pl.BlockSpec((CW, H), lambda i: (0, 0)),
        ],
        out_shape=[
            jax.ShapeDtypeStruct((NR, H), jnp.float32),
            jax.ShapeDtypeStruct((CW, H), jnp.float32),
        ],
        scratch_shapes=[
            pltpu.VMEM((CW, H), jnp.float32),
        ],
    )(g_n, w0, b0, w1, b1, c_n2a_t)


def _attr_stage_body(with_enc, with_leaky, *refs):
    if with_enc:
        (g, w0, b0, w1, b1, agg, cnt, ws, wn, bb, gamma, beta, x_enc_out,
         x_out) = refs
        h = _leaky(_dot(g[...], w0[...]) + b0[...])
        h = _leaky(_dot(h, w1[...]) + b1[...])
        x_enc_out[...] = h
    else:
        (g, agg, cnt, ws, wn, bb, gamma, beta, x_out) = refs
        h = g[...]
    cn = jnp.sum(cnt[...], axis=1, keepdims=True)[:N_ATTR]
    aggr = agg[...] / jnp.maximum(cn, 1.0)
    pre = _dot(h, ws[...]) + _dot(aggr, wn[...]) + bb[...]
    mu = jnp.mean(pre, axis=0, keepdims=True)
    var = jnp.mean((pre - mu) ** 2, axis=0, keepdims=True)
    y = (pre - mu) * lax.rsqrt(var + EPS) * gamma[...] + beta[...]
    if with_leaky:
        y = _leaky(y)
    x_out[...] = y


def _attr_stage(with_enc, with_leaky, args):
    n_in = len(args)
    n_out = 2 if with_enc else 1
    full = lambda s: pl.BlockSpec(s, lambda: (0, 0))
    in_specs = [full(a.shape) for a in args]
    return pl.pallas_call(
        functools.partial(_attr_stage_body, with_enc, with_leaky),
        grid=(),
        in_specs=in_specs,
        out_specs=[full((N_ATTR, H))] * n_out,
        out_shape=[jax.ShapeDtypeStruct((N_ATTR, H), jnp.float32)] * n_out,
    )(*args)


def _name_conv_body(x, c, xa, ws, wn, bb, pre_out, stats_out, s1, s2):
    i = pl.program_id(0)
    aggr = jnp.zeros((BN, H), jnp.float32)
    rs = jnp.zeros((BN, 1), jnp.float32)
    for k in range(CW // 128):
        ck = c[0, k]
        aggr += _dot(ck, xa[pl.ds(k * 128, 128), :])
        rs += jnp.sum(ck, axis=1, keepdims=True)
    aggr = aggr / jnp.maximum(rs, 1.0)
    pre = _dot(x[...], ws[...]) + _dot(aggr, wn[...]) + bb[...]
    pre_out[...] = pre

    @pl.when(i == 0)
    def _():
        s1[...] = jnp.zeros_like(s1)
        s2[...] = jnp.zeros_like(s2)

    row = i * BN + lax.broadcasted_iota(jnp.int32, (BN, 1), 0)
    pm = jnp.where(row < N_NAME, pre, 0.0)
    s1[...] += jnp.sum(pm, axis=0, keepdims=True)
    s2[...] += jnp.sum(pm * pm, axis=0, keepdims=True)

    @pl.when(i == NBLK - 1)
    def _():
        stats_out[0:1, :] = s1[...]
        stats_out[1:2, :] = s2[...]


def _name_conv(x_n, c_a2n, x_a, ws, wn, bb):
    return pl.pallas_call(
        _name_conv_body,
        grid=(NBLK,),
        in_specs=[
            pl.BlockSpec((BN, H), lambda i: (i, 0)),
            pl.BlockSpec((1, CW // 128, CH, 128), lambda i: (i, 0, 0, 0)),
            pl.BlockSpec((CW, H), lambda i: (0, 0)),
            pl.BlockSpec((H, H), lambda i: (0, 0)),
            pl.BlockSpec((H, H), lambda i: (0, 0)),
            pl.BlockSpec((1, H), lambda i: (0, 0)),
        ],
        out_specs=[
            pl.BlockSpec((BN, H), lambda i: (i, 0)),
            pl.BlockSpec((2, H), lambda i: (0, 0)),
        ],
        out_shape=[
            jax.ShapeDtypeStruct((NR, H), jnp.float32),
            jax.ShapeDtypeStruct((2, H), jnp.float32),
        ],
        scratch_shapes=[
            pltpu.VMEM((1, H), jnp.float32),
            pltpu.VMEM((1, H), jnp.float32),
        ],
    )(x_n, c_a2n, x_a, ws, wn, bb)


def _bn_accum_body(pre, stats, gamma, beta, c, x_out, a_out, acc):
    i = pl.program_id(0)
    mu = stats[0:1, :] / N_NAME
    var = stats[1:2, :] / N_NAME - mu * mu
    y = (pre[...] - mu) * lax.rsqrt(var + EPS) * gamma[...] + beta[...]
    y = _leaky(y)
    x_out[...] = y

    @pl.when(i == 0)
    def _():
        acc[...] = jnp.zeros_like(acc)

    for k in range(CW // 128):
        acc[pl.ds(k * 128, 128), :] += _tdot(c[0, k], y)

    @pl.when(i == NBLK - 1)
    def _():
        a_out[...] = acc[...]


def _bn_accum(pre_n, stats, gamma, beta, c_n2a_t):
    return pl.pallas_call(
        _bn_accum_body,
        grid=(NBLK,),
        in_specs=[
            pl.BlockSpec((BN, H), lambda i: (i, 0)),
            pl.BlockSpec((2, H), lambda i: (0, 0)),
            pl.BlockSpec((1, H), lambda i: (0, 0)),
            pl.BlockSpec((1, H), lambda i: (0, 0)),
            pl.BlockSpec((1, CW // 128, CH, 128), lambda i: (i, 0, 0, 0)),
        ],
        out_specs=[
            pl.BlockSpec((BN, H), lambda i: (i, 0)),
            pl.BlockSpec((CW, H), lambda i: (0, 0)),
        ],
        out_shape=[
            jax.ShapeDtypeStruct((NR, H), jnp.float32),
            jax.ShapeDtypeStruct((CW, H), jnp.float32),
        ],
        scratch_shapes=[pltpu.VMEM((CW, H), jnp.float32)],
    )(pre_n, stats, gamma, beta, c_n2a_t)


LB = 1024
LBLK = L // LB


def _distmult_body(rows, stats, gamma, beta, xa, out):
    mu = stats[0:1, :] / N_NAME
    var = stats[1:2, :] / N_NAME - mu * mu
    y = (rows[...] - mu) * lax.rsqrt(var + EPS) * gamma[...] + beta[...]
    out[...] = lax.dot_general(y, xa[...], (((1,), (1,)), ((), ())),
                               preferred_element_type=jnp.float32)


def _distmult(rows, stats, gamma, beta, x_a):
    return pl.pallas_call(
        _distmult_body,
        grid=(LBLK,),
        in_specs=[
            pl.BlockSpec((LB, H), lambda i: (i, 0)),
            pl.BlockSpec((2, H), lambda i: (0, 0)),
            pl.BlockSpec((1, H), lambda i: (0, 0)),
            pl.BlockSpec((1, H), lambda i: (0, 0)),
            pl.BlockSpec((N_ATTR, H), lambda i: (0, 0)),
        ],
        out_specs=pl.BlockSpec((LB, N_ATTR), lambda i: (i, 0)),
        out_shape=jax.ShapeDtypeStruct((L, N_ATTR), jnp.float32),
    )(rows, stats, gamma, beta, x_a)


def _pred_body(p2, out):
    v = p2[...]
    out[...] = jnp.concatenate([v] * 8, axis=0)


def _pred_tile(p2):
    return pl.pallas_call(
        _pred_body,
        grid=(8,),
        in_specs=[pl.BlockSpec((N_ATTR, 128), lambda i: (0, 0))],
        out_specs=pl.BlockSpec((8 * N_ATTR, 128), lambda i: (i, 0)),
        out_shape=jax.ShapeDtypeStruct((64 * N_ATTR, 128), jnp.int32),
    )(p2)


def kernel(params, node_feature_name, node_feature_attr, edge_src_n2a,
           edge_dst_n2a, edge_src_a2n, edge_dst_a2n, edge_label_src,
           edge_label_dst, node_label_attr):
    p = params
    r = lambda v: jnp.reshape(v, (1, H))

    idx_n = jnp.concatenate(
        [node_feature_name[:, 0], jnp.zeros((NR - N_NAME,), jnp.int32)])
    g_n = jnp.take(p['emb_name'], idx_n, axis=0)
    g_a = jnp.take(p['emb_attr'], node_feature_attr[:, 0], axis=0)
    ca_flat, cbt_flat, cnt_raw = _build_counts(
        edge_dst_a2n, edge_src_a2n, edge_src_n2a, edge_dst_n2a)
    c_a2n = jnp.reshape(ca_flat, (NCH, CW // 128, CH, 128))
    c_n2a_t = jnp.reshape(cbt_flat, (NCH, CW // 128, CH, 128))
    cnt_a = jnp.reshape(cnt_raw, (CW, NS))
    pad_a = lambda v: jnp.pad(v, ((0, CW - N_ATTR), (0, 0)))

    x0_n, a0_a = _enc_name(
        g_n, p['mlp_name_W0'], r(p['mlp_name_b0']),
        p['mlp_name_W1'], r(p['mlp_name_b1']), c_n2a_t)
    x0_a, x1_a = _attr_stage(True, True, (
        g_a, p['mlp_attr_W0'], r(p['mlp_attr_b0']),
        p['mlp_attr_W1'], r(p['mlp_attr_b1']),
        a0_a[:N_ATTR], cnt_a,
        p['conv0_n2a_Wself'], p['conv0_n2a_Wneigh'], r(p['conv0_n2a_b']),
        r(p['bn0_attr_gamma']), r(p['bn0_attr_beta'])))
    pre_n1, stats1 = _name_conv(
        x0_n, c_a2n, pad_a(x0_a),
        p['conv0_a2n_Wself'], p['conv0_a2n_Wneigh'], r(p['conv0_a2n_b']))
    x1_n, a1_a = _bn_accum(pre_n1, stats1, r(p['bn0_name_gamma']),
                           r(p['bn0_name_beta']), c_n2a_t)
    (x2_a,) = _attr_stage(False, False, (
        x1_a, a1_a[:N_ATTR], cnt_a,
        p['conv1_n2a_Wself'], p['conv1_n2a_Wneigh'], r(p['conv1_n2a_b']),
        r(p['bn1_attr_gamma']), r(p['bn1_attr_beta'])))
    pre_n2, stats2 = _name_conv(
        x1_n, c_a2n, pad_a(x1_a),
        p['conv1_a2n_Wself'], p['conv1_a2n_Wneigh'], r(p['conv1_a2n_b']))

    rows = jnp.take(pre_n2, edge_label_src, axis=0)
    out = _distmult(rows, stats2, r(p['bn1_name_gamma']),
                    r(p['bn1_name_beta']), x2_a)

    p2 = jnp.reshape(jnp.tile(node_label_attr, (128,)), (N_ATTR, 128))
    pred = jnp.reshape(_pred_tile(p2), (L * N_ATTR,))
    return (out, edge_label_dst, pred)
